# Initial kernel scaffold; baseline (speedup 1.0000x reference)
import jax
import jax.numpy as jnp
from jax.experimental import pallas as pl


def kernel(node_attrs, node_feats, edge_attrs, edge_feats, W_skip, W_up, W_r1, W_r2, W_r3, W_r4, W_lin0, W_lin1, W_msg0, W_msg1, W_out0, W_out1, edge_index, species):
    raise NotImplementedError("write your pallas kernel here")


# speedup vs baseline: 8.6567x; 8.6567x over previous
import functools

import jax
import jax.numpy as jnp
from jax import lax
from jax.experimental import pallas as pl
from jax.experimental.pallas import tpu as pltpu
from jax.experimental.pallas import tpu_sc as plsc

F = 128
S = 10


def _silu(x):
    return x * (1.0 / (1.0 + jnp.exp(-x)))


def _node_pre_body(nf_ref, na_ref, wup_ref, wskipT_ref, h0_ref, skip0_ref):
    x = nf_ref[...]
    h0_ref[...] = jnp.dot(x, wup_ref[...], preferred_element_type=jnp.float32) * (
        F ** -0.5
    )
    a = na_ref[...]
    acc = jnp.zeros(x.shape, jnp.float32)
    for v in range(S):
        acc = acc + a[:, v : v + 1] * jnp.dot(
            x, wskipT_ref[v], preferred_element_type=jnp.float32
        )
    skip0_ref[...] = acc * ((F * S) ** -0.5)


def _node_pre(node_feats, node_attrs, W_up, W_skipT, *, interpret=False):
    N = node_feats.shape[0]
    NB = 1000
    return pl.pallas_call(
        _node_pre_body,
        grid=(N // NB,),
        in_specs=[
            pl.BlockSpec((NB, F), lambda i: (i, 0)),
            pl.BlockSpec((NB, S), lambda i: (i, 0)),
            pl.BlockSpec((F, F), lambda i: (0, 0)),
            pl.BlockSpec((S, F, F), lambda i: (0, 0, 0)),
        ],
        out_specs=[
            pl.BlockSpec((NB, F), lambda i: (i, 0)),
            pl.BlockSpec((NB, F), lambda i: (i, 0)),
        ],
        out_shape=[
            jax.ShapeDtypeStruct((N, F), jnp.float32),
            jax.ShapeDtypeStruct((N, F), jnp.float32),
        ],
        interpret=interpret,
    )(node_feats, node_attrs, W_up, W_skipT)


def _edge_pre_body(ef_ref, ea_ref, s0_ref, wr1_ref, wr2_ref, wr3_ref, wr4_ref, m4_ref):
    x = ef_ref[...]
    w = _silu(jnp.dot(x, wr1_ref[...], preferred_element_type=jnp.float32) * (8.0 ** -0.5))
    w = _silu(jnp.dot(w, wr2_ref[...], preferred_element_type=jnp.float32) * (64.0 ** -0.5))
    w = _silu(jnp.dot(w, wr3_ref[...], preferred_element_type=jnp.float32) * (64.0 ** -0.5))
    w = jnp.dot(w, wr4_ref[...], preferred_element_type=jnp.float32) * (64.0 ** -0.5)
    s = s0_ref[...]
    y = ea_ref[...]
    m4_ref[0] = w[:, :F] * s * y[:, 0:1]
    b = w[:, F:] * s
    m4_ref[1] = b * y[:, 1:2]
    m4_ref[2] = b * y[:, 2:3]
    m4_ref[3] = b * y[:, 3:4]


def _edge_pre(edge_feats, edge_attrs, s0, W_r1, W_r2, W_r3, W_r4b, *, interpret=False):
    E = edge_feats.shape[0]
    EB = 2000
    return pl.pallas_call(
        _edge_pre_body,
        grid=(E // EB,),
        in_specs=[
            pl.BlockSpec((EB, 8), lambda i: (i, 0)),
            pl.BlockSpec((EB, 4), lambda i: (i, 0)),
            pl.BlockSpec((EB, F), lambda i: (i, 0)),
            pl.BlockSpec((8, 64), lambda i: (0, 0)),
            pl.BlockSpec((64, 64), lambda i: (0, 0)),
            pl.BlockSpec((64, 64), lambda i: (0, 0)),
            pl.BlockSpec((64, 2 * F), lambda i: (0, 0)),
        ],
        out_specs=pl.BlockSpec((4, EB, F), lambda i: (0, i, 0)),
        out_shape=jax.ShapeDtypeStruct((4, E, F), jnp.float32),
        interpret=interpret,
    )(edge_feats, edge_attrs, s0, W_r1, W_r2, W_r3, W_r4b)


def _node_post_body(
    msg4_ref, na_ref, wl0_ref, wl1_ref, wm0T_ref, wm1T_ref, wo0_ref, wo1_ref,
    f0_ref, fx_ref, fy_ref, fz_ref,
):
    c1 = ((2 * F) ** -0.5) * 0.25
    c2 = (F * S) ** -0.5
    c3 = F ** -0.5
    a = na_ref[...]

    def species_tp(t, wT_ref):
        acc = jnp.zeros(t.shape, jnp.float32)
        for v in range(S):
            acc = acc + a[:, v : v + 1] * jnp.dot(
                t, wT_ref[v], preferred_element_type=jnp.float32
            )
        return acc * c2

    t0 = jnp.dot(msg4_ref[0], wl0_ref[...], preferred_element_type=jnp.float32) * c1
    o0 = species_tp(t0, wm0T_ref)
    f0_ref[...] = jnp.dot(o0, wo0_ref[...], preferred_element_type=jnp.float32) * c3
    for i, out_ref in ((1, fx_ref), (2, fy_ref), (3, fz_ref)):
        t = jnp.dot(msg4_ref[i], wl1_ref[...], preferred_element_type=jnp.float32) * c1
        o = species_tp(t, wm1T_ref)
        out_ref[...] = jnp.dot(o, wo1_ref[...], preferred_element_type=jnp.float32) * c3


def _node_post(msg4, node_attrs, W_lin0f, W_lin1f, W_msg0T, W_msg1T, W_out0, W_out1,
               *, interpret=False):
    N = node_attrs.shape[0]
    NB = 1000
    return pl.pallas_call(
        _node_post_body,
        grid=(N // NB,),
        in_specs=[
            pl.BlockSpec((4, NB, F), lambda i: (0, i, 0)),
            pl.BlockSpec((NB, S), lambda i: (i, 0)),
            pl.BlockSpec((F, F), lambda i: (0, 0)),
            pl.BlockSpec((F, F), lambda i: (0, 0)),
            pl.BlockSpec((S, F, F), lambda i: (0, 0, 0)),
            pl.BlockSpec((S, F, F), lambda i: (0, 0, 0)),
            pl.BlockSpec((F, F), lambda i: (0, 0)),
            pl.BlockSpec((F, F), lambda i: (0, 0)),
        ],
        out_specs=[pl.BlockSpec((NB, F), lambda i: (i, 0)) for _ in range(4)],
        out_shape=[jax.ShapeDtypeStruct((N, F), jnp.float32) for _ in range(4)],
        interpret=interpret,
    )(msg4, node_attrs, W_lin0f, W_lin1f, W_msg0T, W_msg1T, W_out0, W_out1)


def _sc_gather(h0, snd):
    N = h0.shape[0]
    E = snd.shape[0]
    NW = 32
    e_per_w = E // NW
    CH = 400
    mesh = plsc.VectorSubcoreMesh(core_axis_name="c", subcore_axis_name="s")

    @functools.partial(
        pl.kernel,
        out_type=jax.ShapeDtypeStruct((E, F), jnp.float32),
        mesh=mesh,
        scratch_types=[
            pltpu.VMEM((CH,), jnp.int32),
            pltpu.VMEM((CH, F), jnp.float32),
            pltpu.SemaphoreType.DMA,
        ],
    )
    def gk(h0_hbm, snd_hbm, out_hbm, idx_v, rows_v, sem):
        wid = lax.axis_index("s") * 2 + lax.axis_index("c")
        base = wid * e_per_w

        def body(i, carry):
            off = base + i * CH
            pltpu.sync_copy(snd_hbm.at[pl.ds(off, CH)], idx_v)
            pltpu.async_copy(h0_hbm.at[idx_v], rows_v, sem).wait()
            pltpu.sync_copy(rows_v, out_hbm.at[pl.ds(off, CH)])
            return carry

        lax.fori_loop(0, e_per_w // CH, body, 0)

    return gk(h0, snd)


def _sc_scatter(m4, rcv, zeros_nf):
    E = rcv.shape[0]
    Np = zeros_nf.shape[0]
    CH = 200
    n_per_tile = Np // 16
    e_per_tile = E // 16
    mesh = plsc.VectorSubcoreMesh(core_axis_name="c", subcore_axis_name="s")

    @functools.partial(
        pl.kernel,
        out_type=jax.ShapeDtypeStruct((4, Np, F), jnp.float32),
        mesh=mesh,
        scratch_types=[
            pltpu.VMEM_SHARED((Np, F), jnp.float32),
            pltpu.VMEM((CH,), jnp.int32),
            pltpu.VMEM((CH, F), jnp.float32),
        ],
    )
    def sk(m4_hbm, rcv_hbm, zero_hbm, out_hbm, acc_sh, idx_v, upd_v):
        c = lax.axis_index("c")
        s = lax.axis_index("s")
        r0 = s * n_per_tile
        for j in range(2):
            p = c * 2 + j
            pltpu.sync_copy(
                zero_hbm.at[pl.ds(r0, n_per_tile)],
                acc_sh.at[pl.ds(r0, n_per_tile)],
            )
            plsc.subcore_barrier()

            def body(i, carry):
                off = s * e_per_tile + i * CH
                pltpu.sync_copy(rcv_hbm.at[pl.ds(off, CH)], idx_v)
                pltpu.sync_copy(m4_hbm.at[p].at[pl.ds(off, CH)], upd_v)
                pltpu.sync_copy(upd_v, acc_sh.at[idx_v], add=True)
                return carry

            lax.fori_loop(0, e_per_tile // CH, body, 0)
            plsc.subcore_barrier()
            pltpu.sync_copy(
                acc_sh.at[pl.ds(r0, n_per_tile)],
                out_hbm.at[p].at[pl.ds(r0, n_per_tile)],
            )
            plsc.subcore_barrier()

    return sk(m4, rcv, zeros_nf)


def kernel(node_attrs, node_feats, edge_attrs, edge_feats, W_skip, W_up, W_r1,
           W_r2, W_r3, W_r4, W_lin0, W_lin1, W_msg0, W_msg1, W_out0, W_out1,
           edge_index, species):
    N = node_feats.shape[0]
    snd = edge_index[0]
    rcv = edge_index[1]

    h0, skip0 = _node_pre(node_feats, node_attrs, W_up, W_skip.transpose(1, 0, 2))
    s0 = _sc_gather(h0, snd)
    m4 = _edge_pre(edge_feats, edge_attrs, s0, W_r1, W_r2, W_r3, W_r4[:, : 2 * F])
    Np = ((N // 16 + 7) // 8 * 8) * 16
    msg4 = _sc_scatter(m4, rcv, jnp.zeros((Np, F), jnp.float32))[:, :N]
    f0, fx, fy, fz = _node_post(
        msg4, node_attrs, W_lin0[:F], W_lin1[:F],
        W_msg0.transpose(1, 0, 2), W_msg1.transpose(1, 0, 2), W_out0, W_out1,
    )
    message = jnp.concatenate(
        [f0, jnp.stack([fx, fy, fz], axis=-1).reshape(N, 3 * F)], axis=1
    )
    skip = jnp.concatenate([skip0, jnp.zeros((N, 3 * F), jnp.float32)], axis=1)
    return message, skip
